# baseline (speedup 1.0000x reference)
import jax
import jax.numpy as jnp
from jax.experimental import pallas as pl
from jax.experimental.pallas import tpu as pltpu

_B = 16384
_H = 1024
_X = 2624
_BLOCK = 1024


def _gate_body(c_ref, h_ref, x_ref, wc_ref, wh_ref, wx_ref, b_ref,
               o0_ref, o1_ref, o2_ref):
    logits = jnp.dot(c_ref[...], wc_ref[...], preferred_element_type=jnp.float32)
    logits += jnp.dot(h_ref[...], wh_ref[...], preferred_element_type=jnp.float32)
    logits += jnp.dot(x_ref[...], wx_ref[...], preferred_element_type=jnp.float32)
    logits += b_ref[...]
    m = jnp.max(logits, axis=1, keepdims=True)
    e = jnp.exp(logits - m)
    s = jnp.sum(e, axis=1, keepdims=True)
    w = e / s
    o0_ref[...] = w[:, 0]
    o1_ref[...] = w[:, 1]
    o2_ref[...] = w[:, 2]


def kernel(c_img, h_t, x_t, W, b):
    wc = W[:, :_H].T
    wh = W[:, _H:2 * _H].T
    wx = W[:, 2 * _H:].T
    b2 = b.reshape(1, 3)
    grid = (_B // _BLOCK,)
    out = pl.pallas_call(
        _gate_body,
        grid=grid,
        in_specs=[
            pl.BlockSpec((_BLOCK, _H), lambda i: (i, 0)),
            pl.BlockSpec((_BLOCK, _H), lambda i: (i, 0)),
            pl.BlockSpec((_BLOCK, _X), lambda i: (i, 0)),
            pl.BlockSpec((_H, 3), lambda i: (0, 0)),
            pl.BlockSpec((_H, 3), lambda i: (0, 0)),
            pl.BlockSpec((_X, 3), lambda i: (0, 0)),
            pl.BlockSpec((1, 3), lambda i: (0, 0)),
        ],
        out_specs=[
            pl.BlockSpec((_BLOCK,), lambda i: (i,)),
            pl.BlockSpec((_BLOCK,), lambda i: (i,)),
            pl.BlockSpec((_BLOCK,), lambda i: (i,)),
        ],
        out_shape=[jax.ShapeDtypeStruct((_B,), jnp.float32)] * 3,
    )(c_img, h_t, x_t, wc, wh, wx, b2)
    return tuple(out)


# speedup vs baseline: 1.0015x; 1.0015x over previous
import jax
import jax.numpy as jnp
from jax.experimental import pallas as pl
from jax.experimental.pallas import tpu as pltpu

_B = 16384
_H = 1024
_X = 2624
_BLOCK = 512


def _gate_body(c_ref, h_ref, x_ref, wc_ref, wh_ref, wx_ref, b_ref,
               o0_ref, o1_ref, o2_ref):
    logits = jnp.dot(c_ref[...], wc_ref[...], preferred_element_type=jnp.float32)
    logits += jnp.dot(h_ref[...], wh_ref[...], preferred_element_type=jnp.float32)
    logits += jnp.dot(x_ref[...], wx_ref[...], preferred_element_type=jnp.float32)
    logits += b_ref[...]
    m = jnp.max(logits, axis=1, keepdims=True)
    e = jnp.exp(logits - m)
    s = jnp.sum(e, axis=1, keepdims=True)
    w = e / s
    o0_ref[...] = w[:, 0]
    o1_ref[...] = w[:, 1]
    o2_ref[...] = w[:, 2]


def kernel(c_img, h_t, x_t, W, b):
    wc = W[:, :_H].T
    wh = W[:, _H:2 * _H].T
    wx = W[:, 2 * _H:].T
    b2 = b.reshape(1, 3)
    grid = (_B // _BLOCK,)
    out = pl.pallas_call(
        _gate_body,
        grid=grid,
        in_specs=[
            pl.BlockSpec((_BLOCK, _H), lambda i: (i, 0)),
            pl.BlockSpec((_BLOCK, _H), lambda i: (i, 0)),
            pl.BlockSpec((_BLOCK, _X), lambda i: (i, 0)),
            pl.BlockSpec((_H, 3), lambda i: (0, 0)),
            pl.BlockSpec((_H, 3), lambda i: (0, 0)),
            pl.BlockSpec((_X, 3), lambda i: (0, 0)),
            pl.BlockSpec((1, 3), lambda i: (0, 0)),
        ],
        out_specs=[
            pl.BlockSpec((_BLOCK,), lambda i: (i,)),
            pl.BlockSpec((_BLOCK,), lambda i: (i,)),
            pl.BlockSpec((_BLOCK,), lambda i: (i,)),
        ],
        out_shape=[jax.ShapeDtypeStruct((_B,), jnp.float32)] * 3,
    )(c_img, h_t, x_t, wc, wh, wx, b2)
    return tuple(out)
